# baseline (speedup 1.0000x reference)
import functools

import jax
import jax.numpy as jnp
from jax import lax
from jax.experimental import pallas as pl
from jax.experimental.pallas import tpu as pltpu
from jax.experimental.pallas import tpu_sc as plsc

NMPAD = 128


def _gather_kernel(n_per_w, n_chunk, num_cores, table_hbm, idx_hbm, out_hbm,
                   idx_v, rows_v, sem):
    wid = lax.axis_index("s") * num_cores + lax.axis_index("c")
    base = wid * n_per_w
    for c in range(n_per_w // n_chunk):
        off = base + c * n_chunk
        pltpu.sync_copy(idx_hbm.at[pl.ds(off, n_chunk)], idx_v)
        pltpu.async_copy(table_hbm.at[idx_v], rows_v, sem).wait()
        pltpu.sync_copy(rows_v, out_hbm.at[pl.ds(off, n_chunk)])


def _sc_gather(table, idx):
    n = idx.shape[0]
    d = table.shape[1]
    info = plsc.get_sparse_core_info()
    nw = info.num_cores * info.num_subcores
    n_per_w = n // nw
    n_chunk = min(64, n_per_w)
    mesh = plsc.VectorSubcoreMesh(core_axis_name="c", subcore_axis_name="s")
    kern = pl.kernel(
        functools.partial(_gather_kernel, n_per_w, n_chunk, info.num_cores),
        mesh=mesh,
        out_type=jax.ShapeDtypeStruct((n, d), jnp.float32),
        scratch_types=[
            pltpu.VMEM((n_chunk,), jnp.int32),
            pltpu.VMEM((n_chunk, d), jnp.float32),
            pltpu.SemaphoreType.DMA,
        ],
    )
    return kern(table, idx)


def _top2_routing(logits, nm):
    blk = logits.shape[0]
    col = lax.broadcasted_iota(jnp.int32, (blk, NMPAD), 1)
    neg = jnp.float32(-1e30)
    ml = jnp.where(col < nm, logits, neg)
    m1 = jnp.max(ml, axis=1, keepdims=True)
    i1 = jnp.min(jnp.where(ml >= m1, col, NMPAD), axis=1, keepdims=True)
    oh1 = col == i1
    ml2 = jnp.where(oh1, neg, ml)
    m2 = jnp.max(ml2, axis=1, keepdims=True)
    i2 = jnp.min(jnp.where(ml2 >= m2, col, NMPAD), axis=1, keepdims=True)
    oh2 = col == i2
    t = jnp.exp(m2 - m1)
    g1 = 1.0 / (1.0 + t)
    g2 = 1.0 - g1
    zero = jnp.float32(0.0)
    gate = jnp.where(oh1, g1, zero) + jnp.where(oh2, g2, zero)
    wm = jnp.where(oh1 | oh2, jnp.float32(1.0), zero)
    return gate, wm


def _mega_kernel(bq, nq, nm, xe_ref, wq_ref, wk_ref, wv_ref, wg_ref, wo_ref,
                 bo_ref, o_ref, qs_ref, ks_ref, vs_ref, gs_ref, wms_ref,
                 acc_ref):
    t = pl.program_id(0)
    cdims = (((1,), (1,)), ((), ()))

    @pl.when(t < nq)
    def _proj():
        xe = xe_ref[...]
        xb = xe.astype(jnp.bfloat16)
        sl = pl.ds(t * bq, bq)
        qs_ref[sl, :] = jnp.dot(
            xb, wq_ref[...],
            preferred_element_type=jnp.float32).astype(jnp.bfloat16)
        ks_ref[sl, :] = jnp.dot(
            xb, wk_ref[...],
            preferred_element_type=jnp.float32).astype(jnp.bfloat16)
        vs_ref[sl, :] = jnp.dot(
            xb, wv_ref[...],
            preferred_element_type=jnp.float32).astype(jnp.bfloat16)
        logits = jnp.dot(xe, wg_ref[...], preferred_element_type=jnp.float32)
        gate, wm = _top2_routing(logits, nm)
        gs_ref[sl, :] = gate.astype(jnp.bfloat16)
        wms_ref[sl, :] = wm.astype(jnp.bfloat16)

    @pl.when(t >= nq)
    def _flash():
        u = t - nq
        i = u // nq
        j = lax.rem(u, nq)
        q = qs_ref[pl.ds(i * bq, bq), :]
        gate = gs_ref[pl.ds(i * bq, bq), :]
        ks = ks_ref[pl.ds(j * bq, bq), :]
        vs = vs_ref[pl.ds(j * bq, bq), :]
        wms = wms_ref[pl.ds(j * bq, bq), :]

        @pl.when(j < i)
        def _():
            s = lax.dot_general(q, ks, cdims,
                                preferred_element_type=jnp.float32)
            r = lax.dot_general(gate, wms, cdims,
                                preferred_element_type=jnp.float32)
            a = s * r
            pa = jnp.dot(a.astype(jnp.bfloat16), vs,
                         preferred_element_type=jnp.float32)
            acc_ref[...] = jnp.where(j == 0, pa, acc_ref[...] + pa)

        @pl.when(j == i)
        def _():
            s = lax.dot_general(q, ks, cdims,
                                preferred_element_type=jnp.float32)
            r = lax.dot_general(gate, wms, cdims,
                                preferred_element_type=jnp.float32)
            rows = lax.broadcasted_iota(jnp.int32, (bq, bq), 0)
            cols = lax.broadcasted_iota(jnp.int32, (bq, bq), 1)
            a = jnp.where(rows >= cols, s * r, jnp.float32(0.0))
            pa = jnp.dot(a.astype(jnp.bfloat16), vs,
                         preferred_element_type=jnp.float32)
            acc = jnp.where(i == 0, pa, acc_ref[...] + pa)
            o_ref[...] = (jnp.dot(acc.astype(jnp.bfloat16), wo_ref[...],
                                  preferred_element_type=jnp.float32)
                          + bo_ref[...])


def kernel(x, emb_table, Wq, Wk, Wv, Wg, Wo, bo):
    b, s = x.shape
    e = emb_table.shape[1]
    h = Wq.shape[1]
    nm = Wg.shape[1]
    o = Wo.shape[1]
    bq = 512
    nq = s // bq
    wgp = jnp.pad(Wg, ((0, 0), (0, NMPAD - nm)))
    kern = pl.pallas_call(
        functools.partial(_mega_kernel, bq, nq, nm),
        grid=(nq + nq * nq,),
        in_specs=[
            pl.BlockSpec((bq, e), lambda t: (jnp.minimum(t, nq - 1), 0)),
            pl.BlockSpec((e, h), lambda t: (0, 0)),
            pl.BlockSpec((e, h), lambda t: (0, 0)),
            pl.BlockSpec((e, h), lambda t: (0, 0)),
            pl.BlockSpec((e, NMPAD), lambda t: (0, 0)),
            pl.BlockSpec((h, o), lambda t: (0, 0)),
            pl.BlockSpec((1, o), lambda t: (0, 0)),
        ],
        out_specs=pl.BlockSpec(
            (bq, o), lambda t: (jnp.where(t < nq, 0, (t - nq) // nq), 0)),
        out_shape=jax.ShapeDtypeStruct((s, o), jnp.float32),
        scratch_shapes=[
            pltpu.VMEM((s, h), jnp.bfloat16),
            pltpu.VMEM((s, h), jnp.bfloat16),
            pltpu.VMEM((s, h), jnp.bfloat16),
            pltpu.VMEM((s, NMPAD), jnp.bfloat16),
            pltpu.VMEM((s, NMPAD), jnp.bfloat16),
            pltpu.VMEM((bq, h), jnp.float32),
        ],
    )
    wqb = Wq.astype(jnp.bfloat16)
    wkb = Wk.astype(jnp.bfloat16)
    wvb = Wv.astype(jnp.bfloat16)
    wob = Wo.astype(jnp.bfloat16)
    bo2 = bo.reshape(1, o)
    xes = [_sc_gather(emb_table, x[bb].astype(jnp.int32)) for bb in range(b)]
    outs = [kern(xes[bb], wqb, wkb, wvb, wgp, wob, bo2) for bb in range(b)]
    return jnp.stack(outs, axis=0)


# speedup vs baseline: 1.1183x; 1.1183x over previous
import functools

import jax
import jax.numpy as jnp
from jax import lax
from jax.experimental import pallas as pl
from jax.experimental.pallas import tpu as pltpu
from jax.experimental.pallas import tpu_sc as plsc

NMPAD = 128


def _gather_kernel(n_per_w, n_chunk, num_cores, table_hbm, idx_hbm, out_hbm,
                   idx_v, rows_v, sem):
    wid = lax.axis_index("s") * num_cores + lax.axis_index("c")
    base = wid * n_per_w
    for c in range(n_per_w // n_chunk):
        off = base + c * n_chunk
        pltpu.sync_copy(idx_hbm.at[pl.ds(off, n_chunk)], idx_v)
        pltpu.async_copy(table_hbm.at[idx_v], rows_v, sem).wait()
        pltpu.sync_copy(rows_v, out_hbm.at[pl.ds(off, n_chunk)])


def _sc_gather(table, idx):
    n = idx.shape[0]
    d = table.shape[1]
    info = plsc.get_sparse_core_info()
    nw = info.num_cores * info.num_subcores
    n_per_w = n // nw
    n_chunk = min(64, n_per_w)
    mesh = plsc.VectorSubcoreMesh(core_axis_name="c", subcore_axis_name="s")
    kern = pl.kernel(
        functools.partial(_gather_kernel, n_per_w, n_chunk, info.num_cores),
        mesh=mesh,
        out_type=jax.ShapeDtypeStruct((n, d), jnp.float32),
        scratch_types=[
            pltpu.VMEM((n_chunk,), jnp.int32),
            pltpu.VMEM((n_chunk, d), jnp.float32),
            pltpu.SemaphoreType.DMA,
        ],
    )
    return kern(table, idx)


def _top2_routing(logits, nm):
    blk = logits.shape[0]
    col = lax.broadcasted_iota(jnp.int32, (blk, NMPAD), 1)
    neg = jnp.float32(-1e30)
    ml = jnp.where(col < nm, logits, neg)
    m1 = jnp.max(ml, axis=1, keepdims=True)
    i1 = jnp.min(jnp.where(ml >= m1, col, NMPAD), axis=1, keepdims=True)
    oh1 = col == i1
    ml2 = jnp.where(oh1, neg, ml)
    m2 = jnp.max(ml2, axis=1, keepdims=True)
    i2 = jnp.min(jnp.where(ml2 >= m2, col, NMPAD), axis=1, keepdims=True)
    oh2 = col == i2
    t = jnp.exp(m2 - m1)
    g1 = 1.0 / (1.0 + t)
    g2 = 1.0 - g1
    zero = jnp.float32(0.0)
    gate = jnp.where(oh1, g1, zero) + jnp.where(oh2, g2, zero)
    wm = jnp.where(oh1 | oh2, jnp.float32(1.0), zero)
    return gate, wm


def _mega_kernel(bq, nq, nm, xe_ref, wq_ref, wk_ref, wv_ref, wg_ref, wo_ref,
                 bo_ref, o_ref, qs_ref, ks_ref, vs_ref, gs_ref, wms_ref,
                 acc_ref):
    t = pl.program_id(1)
    cdims = (((1,), (1,)), ((), ()))

    @pl.when(t < nq)
    def _proj():
        xe = xe_ref[0]
        xb = xe.astype(jnp.bfloat16)
        sl = pl.ds(t * bq, bq)
        qs_ref[sl, :] = jnp.dot(
            xb, wq_ref[...],
            preferred_element_type=jnp.float32).astype(jnp.bfloat16)
        ks_ref[sl, :] = jnp.dot(
            xb, wk_ref[...],
            preferred_element_type=jnp.float32).astype(jnp.bfloat16)
        vs_ref[sl, :] = jnp.dot(
            xb, wv_ref[...],
            preferred_element_type=jnp.float32).astype(jnp.bfloat16)
        logits = jnp.dot(xe, wg_ref[...], preferred_element_type=jnp.float32)
        gate, wm = _top2_routing(logits, nm)
        gs_ref[sl, :] = gate.astype(jnp.bfloat16)
        wms_ref[sl, :] = wm.astype(jnp.bfloat16)

    @pl.when(t >= nq)
    def _flash():
        u = t - nq
        i = u // nq
        j = lax.rem(u, nq)
        q = qs_ref[pl.ds(i * bq, bq), :]
        gate = gs_ref[pl.ds(i * bq, bq), :]
        ks = ks_ref[pl.ds(j * bq, bq), :]
        vs = vs_ref[pl.ds(j * bq, bq), :]
        wms = wms_ref[pl.ds(j * bq, bq), :]

        @pl.when(j < i)
        def _():
            s = lax.dot_general(q, ks, cdims,
                                preferred_element_type=jnp.float32)
            r = lax.dot_general(gate, wms, cdims,
                                preferred_element_type=jnp.float32)
            a = s * r
            pa = jnp.dot(a.astype(jnp.bfloat16), vs,
                         preferred_element_type=jnp.float32)
            acc_ref[...] = jnp.where(j == 0, pa, acc_ref[...] + pa)

        @pl.when(j == i)
        def _():
            s = lax.dot_general(q, ks, cdims,
                                preferred_element_type=jnp.float32)
            r = lax.dot_general(gate, wms, cdims,
                                preferred_element_type=jnp.float32)
            rows = lax.broadcasted_iota(jnp.int32, (bq, bq), 0)
            cols = lax.broadcasted_iota(jnp.int32, (bq, bq), 1)
            a = jnp.where(rows >= cols, s * r, jnp.float32(0.0))
            pa = jnp.dot(a.astype(jnp.bfloat16), vs,
                         preferred_element_type=jnp.float32)
            acc = jnp.where(i == 0, pa, acc_ref[...] + pa)
            o_ref[0] = (jnp.dot(acc.astype(jnp.bfloat16), wo_ref[...],
                                preferred_element_type=jnp.float32)
                        + bo_ref[...])


def kernel(x, emb_table, Wq, Wk, Wv, Wg, Wo, bo):
    b, s = x.shape
    e = emb_table.shape[1]
    h = Wq.shape[1]
    nm = Wg.shape[1]
    o = Wo.shape[1]
    bq = 512
    nq = s // bq
    wgp = jnp.pad(Wg, ((0, 0), (0, NMPAD - nm)))
    kern = pl.pallas_call(
        functools.partial(_mega_kernel, bq, nq, nm),
        grid=(b, nq + nq * nq),
        in_specs=[
            pl.BlockSpec((1, bq, e),
                         lambda b_, t: (b_, jnp.minimum(t, nq - 1), 0)),
            pl.BlockSpec((e, h), lambda b_, t: (0, 0)),
            pl.BlockSpec((e, h), lambda b_, t: (0, 0)),
            pl.BlockSpec((e, h), lambda b_, t: (0, 0)),
            pl.BlockSpec((e, NMPAD), lambda b_, t: (0, 0)),
            pl.BlockSpec((h, o), lambda b_, t: (0, 0)),
            pl.BlockSpec((1, o), lambda b_, t: (0, 0)),
        ],
        out_specs=pl.BlockSpec(
            (1, bq, o),
            lambda b_, t: (b_, jnp.where(t < nq, 0, (t - nq) // nq), 0)),
        out_shape=jax.ShapeDtypeStruct((b, s, o), jnp.float32),
        scratch_shapes=[
            pltpu.VMEM((s, h), jnp.bfloat16),
            pltpu.VMEM((s, h), jnp.bfloat16),
            pltpu.VMEM((s, h), jnp.bfloat16),
            pltpu.VMEM((s, NMPAD), jnp.bfloat16),
            pltpu.VMEM((s, NMPAD), jnp.bfloat16),
            pltpu.VMEM((bq, h), jnp.float32),
        ],
    )
    idx = x.reshape(-1).astype(jnp.int32)
    xe = _sc_gather(emb_table, idx)
    out = kern(xe.reshape(b, s, e), Wq.astype(jnp.bfloat16),
               Wk.astype(jnp.bfloat16), Wv.astype(jnp.bfloat16), wgp,
               Wo.astype(jnp.bfloat16), bo.reshape(1, o))
    return out


# speedup vs baseline: 1.1194x; 1.0010x over previous
import functools

import jax
import jax.numpy as jnp
from jax import lax
from jax.experimental import pallas as pl
from jax.experimental.pallas import tpu as pltpu
from jax.experimental.pallas import tpu_sc as plsc

NMPAD = 128


def _gather_kernel(n_per_w, n_chunk, num_cores, table_hbm, idx_hbm, out_hbm,
                   idx_v, rows0, rows1, g0, g1, s0, s1):
    wid = lax.axis_index("s") * num_cores + lax.axis_index("c")
    base = wid * n_per_w
    nc = n_per_w // n_chunk
    rows = (rows0, rows1)
    gsem = (g0, g1)
    ssem = (s0, s1)
    pltpu.sync_copy(idx_hbm.at[pl.ds(base, n_per_w)], idx_v)

    def issue_gather(c):
        buf = c % 2
        return pltpu.async_copy(
            table_hbm.at[idx_v.at[pl.ds(c * n_chunk, n_chunk)]],
            rows[buf], gsem[buf])

    gat = issue_gather(0)
    scat = [None, None]
    for c in range(nc):
        buf = c % 2
        gat.wait()
        if c + 1 < nc:
            nbuf = (c + 1) % 2
            if scat[nbuf] is not None:
                scat[nbuf].wait()
            gat = issue_gather(c + 1)
        scat[buf] = pltpu.async_copy(
            rows[buf], out_hbm.at[pl.ds(base + c * n_chunk, n_chunk)],
            ssem[buf])
    for sc in scat:
        if sc is not None:
            sc.wait()


def _sc_gather(table, idx):
    n = idx.shape[0]
    d = table.shape[1]
    info = plsc.get_sparse_core_info()
    nw = info.num_cores * info.num_subcores
    n_per_w = n // nw
    n_chunk = min(32, n_per_w)
    mesh = plsc.VectorSubcoreMesh(core_axis_name="c", subcore_axis_name="s")
    kern = pl.kernel(
        functools.partial(_gather_kernel, n_per_w, n_chunk, info.num_cores),
        mesh=mesh,
        out_type=jax.ShapeDtypeStruct((n, d), jnp.float32),
        scratch_types=[
            pltpu.VMEM((n_per_w,), jnp.int32),
            pltpu.VMEM((n_chunk, d), jnp.float32),
            pltpu.VMEM((n_chunk, d), jnp.float32),
            pltpu.SemaphoreType.DMA,
            pltpu.SemaphoreType.DMA,
            pltpu.SemaphoreType.DMA,
            pltpu.SemaphoreType.DMA,
        ],
    )
    return kern(table, idx)


def _top2_routing(logits, nm):
    blk = logits.shape[0]
    col = lax.broadcasted_iota(jnp.int32, (blk, NMPAD), 1)
    neg = jnp.float32(-1e30)
    ml = jnp.where(col < nm, logits, neg)
    m1 = jnp.max(ml, axis=1, keepdims=True)
    i1 = jnp.min(jnp.where(ml >= m1, col, NMPAD), axis=1, keepdims=True)
    oh1 = col == i1
    ml2 = jnp.where(oh1, neg, ml)
    m2 = jnp.max(ml2, axis=1, keepdims=True)
    i2 = jnp.min(jnp.where(ml2 >= m2, col, NMPAD), axis=1, keepdims=True)
    oh2 = col == i2
    t = jnp.exp(m2 - m1)
    g1 = 1.0 / (1.0 + t)
    g2 = 1.0 - g1
    zero = jnp.float32(0.0)
    gate = jnp.where(oh1, g1, zero) + jnp.where(oh2, g2, zero)
    wm = jnp.where(oh1 | oh2, jnp.float32(1.0), zero)
    return gate, wm


def _mega_kernel(bq, nq, nm, xe_ref, wq_ref, wk_ref, wv_ref, wg_ref, wo_ref,
                 bo_ref, o_ref, qs_ref, ks_ref, vs_ref, gs_ref, wms_ref,
                 acc_ref):
    t = pl.program_id(1)
    cdims = (((1,), (1,)), ((), ()))

    @pl.when(t < nq)
    def _proj():
        xe = xe_ref[0]
        xb = xe.astype(jnp.bfloat16)
        sl = pl.ds(t * bq, bq)
        qs_ref[sl, :] = jnp.dot(
            xb, wq_ref[...],
            preferred_element_type=jnp.float32).astype(jnp.bfloat16)
        ks_ref[sl, :] = jnp.dot(
            xb, wk_ref[...],
            preferred_element_type=jnp.float32).astype(jnp.bfloat16)
        vs_ref[sl, :] = jnp.dot(
            xb, wv_ref[...],
            preferred_element_type=jnp.float32).astype(jnp.bfloat16)
        logits = jnp.dot(xe, wg_ref[...], preferred_element_type=jnp.float32)
        gate, wm = _top2_routing(logits, nm)
        gs_ref[sl, :] = gate.astype(jnp.bfloat16)
        wms_ref[sl, :] = wm.astype(jnp.bfloat16)

    @pl.when(t >= nq)
    def _flash():
        u = t - nq
        i = u // nq
        j = lax.rem(u, nq)
        q = qs_ref[pl.ds(i * bq, bq), :]
        gate = gs_ref[pl.ds(i * bq, bq), :]
        ks = ks_ref[pl.ds(j * bq, bq), :]
        vs = vs_ref[pl.ds(j * bq, bq), :]
        wms = wms_ref[pl.ds(j * bq, bq), :]

        @pl.when(j < i)
        def _():
            s = lax.dot_general(q, ks, cdims,
                                preferred_element_type=jnp.float32)
            r = lax.dot_general(gate, wms, cdims,
                                preferred_element_type=jnp.float32)
            a = s * r
            pa = jnp.dot(a.astype(jnp.bfloat16), vs,
                         preferred_element_type=jnp.float32)
            acc_ref[...] = jnp.where(j == 0, pa, acc_ref[...] + pa)

        @pl.when(j == i)
        def _():
            s = lax.dot_general(q, ks, cdims,
                                preferred_element_type=jnp.float32)
            r = lax.dot_general(gate, wms, cdims,
                                preferred_element_type=jnp.float32)
            rows = lax.broadcasted_iota(jnp.int32, (bq, bq), 0)
            cols = lax.broadcasted_iota(jnp.int32, (bq, bq), 1)
            a = jnp.where(rows >= cols, s * r, jnp.float32(0.0))
            pa = jnp.dot(a.astype(jnp.bfloat16), vs,
                         preferred_element_type=jnp.float32)
            acc = jnp.where(i == 0, pa, acc_ref[...] + pa)
            o_ref[0] = (jnp.dot(acc.astype(jnp.bfloat16), wo_ref[...],
                                preferred_element_type=jnp.float32)
                        + bo_ref[...])


def kernel(x, emb_table, Wq, Wk, Wv, Wg, Wo, bo):
    b, s = x.shape
    e = emb_table.shape[1]
    h = Wq.shape[1]
    nm = Wg.shape[1]
    o = Wo.shape[1]
    bq = 512
    nq = s // bq
    wgp = jnp.pad(Wg, ((0, 0), (0, NMPAD - nm)))
    kern = pl.pallas_call(
        functools.partial(_mega_kernel, bq, nq, nm),
        grid=(b, nq + nq * nq),
        in_specs=[
            pl.BlockSpec((1, bq, e),
                         lambda b_, t: (b_, jnp.minimum(t, nq - 1), 0)),
            pl.BlockSpec((e, h), lambda b_, t: (0, 0)),
            pl.BlockSpec((e, h), lambda b_, t: (0, 0)),
            pl.BlockSpec((e, h), lambda b_, t: (0, 0)),
            pl.BlockSpec((e, NMPAD), lambda b_, t: (0, 0)),
            pl.BlockSpec((h, o), lambda b_, t: (0, 0)),
            pl.BlockSpec((1, o), lambda b_, t: (0, 0)),
        ],
        out_specs=pl.BlockSpec(
            (1, bq, o),
            lambda b_, t: (b_, jnp.where(t < nq, 0, (t - nq) // nq), 0)),
        out_shape=jax.ShapeDtypeStruct((b, s, o), jnp.float32),
        scratch_shapes=[
            pltpu.VMEM((s, h), jnp.bfloat16),
            pltpu.VMEM((s, h), jnp.bfloat16),
            pltpu.VMEM((s, h), jnp.bfloat16),
            pltpu.VMEM((s, NMPAD), jnp.bfloat16),
            pltpu.VMEM((s, NMPAD), jnp.bfloat16),
            pltpu.VMEM((bq, h), jnp.float32),
        ],
    )
    idx = x.reshape(-1).astype(jnp.int32)
    xe = _sc_gather(emb_table, idx)
    out = kern(xe.reshape(b, s, e), Wq.astype(jnp.bfloat16),
               Wk.astype(jnp.bfloat16), Wv.astype(jnp.bfloat16), wgp,
               Wo.astype(jnp.bfloat16), bo.reshape(1, o))
    return out


# speedup vs baseline: 1.1282x; 1.0078x over previous
import functools

import jax
import jax.numpy as jnp
from jax import lax
from jax.experimental import pallas as pl
from jax.experimental.pallas import tpu as pltpu
from jax.experimental.pallas import tpu_sc as plsc

NMPAD = 128


NBUF = 3


def _gather_kernel(n_per_w, n_chunk, num_cores, table_hbm, idx_hbm, out_hbm,
                   idx_v, *bufs):
    wid = lax.axis_index("s") * num_cores + lax.axis_index("c")
    base = wid * n_per_w
    nc = n_per_w // n_chunk
    rows = bufs[:NBUF]
    gsem = bufs[NBUF:2 * NBUF]
    ssem = bufs[2 * NBUF:]
    pltpu.sync_copy(idx_hbm.at[pl.ds(base, n_per_w)], idx_v)

    def issue_gather(c):
        buf = c % NBUF
        return pltpu.async_copy(
            table_hbm.at[idx_v.at[pl.ds(c * n_chunk, n_chunk)]],
            rows[buf], gsem[buf])

    gat = [None] * NBUF
    scat = [None] * NBUF
    for c in range(min(NBUF, nc)):
        gat[c % NBUF] = issue_gather(c)
    for c in range(nc):
        buf = c % NBUF
        gat[buf].wait()
        scat[buf] = pltpu.async_copy(
            rows[buf], out_hbm.at[pl.ds(base + c * n_chunk, n_chunk)],
            ssem[buf])
        nxt = c + NBUF
        if nxt < nc:
            scat[buf].wait()
            scat[buf] = None
            gat[buf] = issue_gather(nxt)
    for sc in scat:
        if sc is not None:
            sc.wait()


def _sc_gather(table, idx):
    n = idx.shape[0]
    d = table.shape[1]
    info = plsc.get_sparse_core_info()
    nw = info.num_cores * info.num_subcores
    n_per_w = n // nw
    n_chunk = min(32, n_per_w)
    mesh = plsc.VectorSubcoreMesh(core_axis_name="c", subcore_axis_name="s")
    kern = pl.kernel(
        functools.partial(_gather_kernel, n_per_w, n_chunk, info.num_cores),
        mesh=mesh,
        out_type=jax.ShapeDtypeStruct((n, d), jnp.float32),
        scratch_types=(
            [pltpu.VMEM((n_per_w,), jnp.int32)]
            + [pltpu.VMEM((n_chunk, d), jnp.float32) for _ in range(NBUF)]
            + [pltpu.SemaphoreType.DMA for _ in range(2 * NBUF)]
        ),
    )
    return kern(table, idx)


def _top2_routing(logits, nm):
    blk = logits.shape[0]
    col = lax.broadcasted_iota(jnp.int32, (blk, NMPAD), 1)
    neg = jnp.float32(-1e30)
    ml = jnp.where(col < nm, logits, neg)
    m1 = jnp.max(ml, axis=1, keepdims=True)
    i1 = jnp.min(jnp.where(ml >= m1, col, NMPAD), axis=1, keepdims=True)
    oh1 = col == i1
    ml2 = jnp.where(oh1, neg, ml)
    m2 = jnp.max(ml2, axis=1, keepdims=True)
    i2 = jnp.min(jnp.where(ml2 >= m2, col, NMPAD), axis=1, keepdims=True)
    oh2 = col == i2
    t = jnp.exp(m2 - m1)
    g1 = 1.0 / (1.0 + t)
    g2 = 1.0 - g1
    zero = jnp.float32(0.0)
    gate = jnp.where(oh1, g1, zero) + jnp.where(oh2, g2, zero)
    wm = jnp.where(oh1 | oh2, jnp.float32(1.0), zero)
    return gate, wm


def _mega_kernel(bq, nq, nm, xe_ref, wq_ref, wk_ref, wv_ref, wg_ref, wo_ref,
                 bo_ref, o_ref, qs_ref, ks_ref, vs_ref, gs_ref, wms_ref,
                 acc_ref):
    t = pl.program_id(1)
    cdims = (((1,), (1,)), ((), ()))

    @pl.when(t < nq)
    def _proj():
        xe = xe_ref[0]
        xb = xe.astype(jnp.bfloat16)
        sl = pl.ds(t * bq, bq)
        qs_ref[sl, :] = jnp.dot(
            xb, wq_ref[...],
            preferred_element_type=jnp.float32).astype(jnp.bfloat16)
        ks_ref[sl, :] = jnp.dot(
            xb, wk_ref[...],
            preferred_element_type=jnp.float32).astype(jnp.bfloat16)
        vs_ref[sl, :] = jnp.dot(
            xb, wv_ref[...],
            preferred_element_type=jnp.float32).astype(jnp.bfloat16)
        logits = jnp.dot(xe, wg_ref[...], preferred_element_type=jnp.float32)
        gate, wm = _top2_routing(logits, nm)
        gs_ref[sl, :] = gate.astype(jnp.bfloat16)
        wms_ref[sl, :] = wm.astype(jnp.bfloat16)

    @pl.when(t >= nq)
    def _flash():
        u = t - nq
        i = u // nq
        j = lax.rem(u, nq)
        q = qs_ref[pl.ds(i * bq, bq), :]
        gate = gs_ref[pl.ds(i * bq, bq), :]
        ks = ks_ref[pl.ds(j * bq, bq), :]
        vs = vs_ref[pl.ds(j * bq, bq), :]
        wms = wms_ref[pl.ds(j * bq, bq), :]

        @pl.when(j < i)
        def _():
            s = lax.dot_general(q, ks, cdims,
                                preferred_element_type=jnp.float32)
            r = lax.dot_general(gate, wms, cdims,
                                preferred_element_type=jnp.float32)
            a = s * r
            pa = jnp.dot(a.astype(jnp.bfloat16), vs,
                         preferred_element_type=jnp.float32)
            acc_ref[...] = jnp.where(j == 0, pa, acc_ref[...] + pa)

        @pl.when(j == i)
        def _():
            s = lax.dot_general(q, ks, cdims,
                                preferred_element_type=jnp.float32)
            r = lax.dot_general(gate, wms, cdims,
                                preferred_element_type=jnp.float32)
            rows = lax.broadcasted_iota(jnp.int32, (bq, bq), 0)
            cols = lax.broadcasted_iota(jnp.int32, (bq, bq), 1)
            a = jnp.where(rows >= cols, s * r, jnp.float32(0.0))
            pa = jnp.dot(a.astype(jnp.bfloat16), vs,
                         preferred_element_type=jnp.float32)
            acc = jnp.where(i == 0, pa, acc_ref[...] + pa)
            o_ref[0] = (jnp.dot(acc.astype(jnp.bfloat16), wo_ref[...],
                                preferred_element_type=jnp.float32)
                        + bo_ref[...])


def kernel(x, emb_table, Wq, Wk, Wv, Wg, Wo, bo):
    b, s = x.shape
    e = emb_table.shape[1]
    h = Wq.shape[1]
    nm = Wg.shape[1]
    o = Wo.shape[1]
    bq = 512
    nq = s // bq
    wgp = jnp.pad(Wg, ((0, 0), (0, NMPAD - nm)))
    kern = pl.pallas_call(
        functools.partial(_mega_kernel, bq, nq, nm),
        grid=(b, nq + nq * nq),
        in_specs=[
            pl.BlockSpec((1, bq, e),
                         lambda b_, t: (b_, jnp.minimum(t, nq - 1), 0)),
            pl.BlockSpec((e, h), lambda b_, t: (0, 0)),
            pl.BlockSpec((e, h), lambda b_, t: (0, 0)),
            pl.BlockSpec((e, h), lambda b_, t: (0, 0)),
            pl.BlockSpec((e, NMPAD), lambda b_, t: (0, 0)),
            pl.BlockSpec((h, o), lambda b_, t: (0, 0)),
            pl.BlockSpec((1, o), lambda b_, t: (0, 0)),
        ],
        out_specs=pl.BlockSpec(
            (1, bq, o),
            lambda b_, t: (b_, jnp.where(t < nq, 0, (t - nq) // nq), 0)),
        out_shape=jax.ShapeDtypeStruct((b, s, o), jnp.float32),
        scratch_shapes=[
            pltpu.VMEM((s, h), jnp.bfloat16),
            pltpu.VMEM((s, h), jnp.bfloat16),
            pltpu.VMEM((s, h), jnp.bfloat16),
            pltpu.VMEM((s, NMPAD), jnp.bfloat16),
            pltpu.VMEM((s, NMPAD), jnp.bfloat16),
            pltpu.VMEM((bq, h), jnp.float32),
        ],
    )
    idx = x.reshape(-1).astype(jnp.int32)
    xe = _sc_gather(emb_table, idx)
    out = kern(xe.reshape(b, s, e), Wq.astype(jnp.bfloat16),
               Wk.astype(jnp.bfloat16), Wv.astype(jnp.bfloat16), wgp,
               Wo.astype(jnp.bfloat16), bo.reshape(1, o))
    return out


# speedup vs baseline: 1.1644x; 1.0321x over previous
import functools

import jax
import jax.numpy as jnp
from jax import lax
from jax.experimental import pallas as pl
from jax.experimental.pallas import tpu as pltpu
from jax.experimental.pallas import tpu_sc as plsc

NMPAD = 128


NBUF = 3


def _gather_kernel(n_per_w, n_chunk, num_cores, table_hbm, idx_hbm, out_hbm,
                   idx_v, *bufs):
    wid = lax.axis_index("s") * num_cores + lax.axis_index("c")
    base = wid * n_per_w
    nc = n_per_w // n_chunk
    rows = bufs[:NBUF]
    gsem = bufs[NBUF:2 * NBUF]
    ssem = bufs[2 * NBUF:]
    pltpu.sync_copy(idx_hbm.at[pl.ds(base, n_per_w)], idx_v)

    def issue_gather(c):
        buf = c % NBUF
        return pltpu.async_copy(
            table_hbm.at[idx_v.at[pl.ds(c * n_chunk, n_chunk)]],
            rows[buf], gsem[buf])

    gat = [None] * NBUF
    scat = [None] * NBUF
    for c in range(min(NBUF, nc)):
        gat[c % NBUF] = issue_gather(c)
    for c in range(nc):
        buf = c % NBUF
        gat[buf].wait()
        scat[buf] = pltpu.async_copy(
            rows[buf], out_hbm.at[pl.ds(base + c * n_chunk, n_chunk)],
            ssem[buf])
        nxt = c + NBUF
        if nxt < nc:
            scat[buf].wait()
            scat[buf] = None
            gat[buf] = issue_gather(nxt)
    for sc in scat:
        if sc is not None:
            sc.wait()


def _sc_gather(table, idx):
    n = idx.shape[0]
    d = table.shape[1]
    info = plsc.get_sparse_core_info()
    nw = info.num_cores * info.num_subcores
    n_per_w = n // nw
    n_chunk = min(32, n_per_w)
    mesh = plsc.VectorSubcoreMesh(core_axis_name="c", subcore_axis_name="s")
    kern = pl.kernel(
        functools.partial(_gather_kernel, n_per_w, n_chunk, info.num_cores),
        mesh=mesh,
        out_type=jax.ShapeDtypeStruct((n, d), jnp.float32),
        scratch_types=(
            [pltpu.VMEM((n_per_w,), jnp.int32)]
            + [pltpu.VMEM((n_chunk, d), jnp.float32) for _ in range(NBUF)]
            + [pltpu.SemaphoreType.DMA for _ in range(2 * NBUF)]
        ),
    )
    return kern(table, idx)


def _top2_routing(logits, nm):
    blk = logits.shape[0]
    col = lax.broadcasted_iota(jnp.int32, (blk, NMPAD), 1)
    neg = jnp.float32(-1e30)
    ml = jnp.where(col < nm, logits, neg)
    m1 = jnp.max(ml, axis=1, keepdims=True)
    i1 = jnp.min(jnp.where(ml >= m1, col, NMPAD), axis=1, keepdims=True)
    oh1 = col == i1
    ml2 = jnp.where(oh1, neg, ml)
    m2 = jnp.max(ml2, axis=1, keepdims=True)
    i2 = jnp.min(jnp.where(ml2 >= m2, col, NMPAD), axis=1, keepdims=True)
    oh2 = col == i2
    t = jnp.exp(m2 - m1)
    g1 = 1.0 / (1.0 + t)
    g2 = 1.0 - g1
    zero = jnp.float32(0.0)
    gate = jnp.where(oh1, g1, zero) + jnp.where(oh2, g2, zero)
    wm = jnp.where(oh1 | oh2, jnp.float32(1.0), zero)
    return gate, wm


def _mega_kernel(bq, nq, nm, xe_ref, wq_ref, wk_ref, wv_ref, wg_ref, wo_ref,
                 bo_ref, o_ref, qs_ref, ks_ref, vs_ref, gs_ref, wms_ref,
                 acc_ref):
    t = pl.program_id(1)
    cdims = (((1,), (1,)), ((), ()))

    @pl.when(t < nq)
    def _proj():
        xe = xe_ref[0]
        xb = xe.astype(jnp.bfloat16)
        sl = pl.ds(t * bq, bq)
        qs_ref[sl, :] = jnp.dot(
            xb, wq_ref[...],
            preferred_element_type=jnp.float32).astype(jnp.bfloat16)
        ks_ref[sl, :] = jnp.dot(
            xb, wk_ref[...],
            preferred_element_type=jnp.float32).astype(jnp.bfloat16)
        vs_ref[sl, :] = jnp.dot(
            xb, wv_ref[...],
            preferred_element_type=jnp.float32).astype(jnp.bfloat16)
        logits = jnp.dot(xe, wg_ref[...], preferred_element_type=jnp.float32)
        gate, wm = _top2_routing(logits, nm)
        gs_ref[sl, :] = gate.astype(jnp.bfloat16)
        wms_ref[sl, :] = wm.astype(jnp.bfloat16)

    @pl.when(t >= nq)
    def _flash():
        u = t - nq
        i = u * 0
        for r in range(1, nq):
            i = i + jnp.where(u >= (r * (r + 1)) // 2, 1, 0)
        j = u - (i * (i + 1)) // 2
        q = qs_ref[pl.ds(i * bq, bq), :]
        gate = gs_ref[pl.ds(i * bq, bq), :]
        ks = ks_ref[pl.ds(j * bq, bq), :]
        vs = vs_ref[pl.ds(j * bq, bq), :]
        wms = wms_ref[pl.ds(j * bq, bq), :]

        @pl.when(j < i)
        def _():
            s = lax.dot_general(q, ks, cdims,
                                preferred_element_type=jnp.float32)
            r = lax.dot_general(gate, wms, cdims,
                                preferred_element_type=jnp.float32)
            a = s * r
            pa = jnp.dot(a.astype(jnp.bfloat16), vs,
                         preferred_element_type=jnp.float32)
            acc_ref[...] = jnp.where(j == 0, pa, acc_ref[...] + pa)

        @pl.when(j == i)
        def _():
            s = lax.dot_general(q, ks, cdims,
                                preferred_element_type=jnp.float32)
            r = lax.dot_general(gate, wms, cdims,
                                preferred_element_type=jnp.float32)
            rows = lax.broadcasted_iota(jnp.int32, (bq, bq), 0)
            cols = lax.broadcasted_iota(jnp.int32, (bq, bq), 1)
            a = jnp.where(rows >= cols, s * r, jnp.float32(0.0))
            pa = jnp.dot(a.astype(jnp.bfloat16), vs,
                         preferred_element_type=jnp.float32)
            acc = jnp.where(i == 0, pa, acc_ref[...] + pa)
            o_ref[0] = (jnp.dot(acc.astype(jnp.bfloat16), wo_ref[...],
                                preferred_element_type=jnp.float32)
                        + bo_ref[...])


def kernel(x, emb_table, Wq, Wk, Wv, Wg, Wo, bo):
    b, s = x.shape
    e = emb_table.shape[1]
    h = Wq.shape[1]
    nm = Wg.shape[1]
    o = Wo.shape[1]
    bq = 512
    nq = s // bq
    wgp = jnp.pad(Wg, ((0, 0), (0, NMPAD - nm)))
    ntri = (nq * (nq + 1)) // 2

    def _row(t):
        u = t - nq
        i = u * 0
        for r in range(1, nq):
            i = i + jnp.where(u >= (r * (r + 1)) // 2, 1, 0)
        return i

    kern = pl.pallas_call(
        functools.partial(_mega_kernel, bq, nq, nm),
        grid=(b, nq + ntri),
        in_specs=[
            pl.BlockSpec((1, bq, e),
                         lambda b_, t: (b_, jnp.minimum(t, nq - 1), 0)),
            pl.BlockSpec((e, h), lambda b_, t: (0, 0)),
            pl.BlockSpec((e, h), lambda b_, t: (0, 0)),
            pl.BlockSpec((e, h), lambda b_, t: (0, 0)),
            pl.BlockSpec((e, NMPAD), lambda b_, t: (0, 0)),
            pl.BlockSpec((h, o), lambda b_, t: (0, 0)),
            pl.BlockSpec((1, o), lambda b_, t: (0, 0)),
        ],
        out_specs=pl.BlockSpec(
            (1, bq, o),
            lambda b_, t: (b_, jnp.where(t < nq, 0, _row(t)), 0)),
        out_shape=jax.ShapeDtypeStruct((b, s, o), jnp.float32),
        scratch_shapes=[
            pltpu.VMEM((s, h), jnp.bfloat16),
            pltpu.VMEM((s, h), jnp.bfloat16),
            pltpu.VMEM((s, h), jnp.bfloat16),
            pltpu.VMEM((s, NMPAD), jnp.bfloat16),
            pltpu.VMEM((s, NMPAD), jnp.bfloat16),
            pltpu.VMEM((bq, h), jnp.float32),
        ],
    )
    idx = x.reshape(-1).astype(jnp.int32)
    xe = _sc_gather(emb_table, idx)
    out = kern(xe.reshape(b, s, e), Wq.astype(jnp.bfloat16),
               Wk.astype(jnp.bfloat16), Wv.astype(jnp.bfloat16), wgp,
               Wo.astype(jnp.bfloat16), bo.reshape(1, o))
    return out


# speedup vs baseline: 1.1717x; 1.0062x over previous
import functools

import jax
import jax.numpy as jnp
from jax import lax
from jax.experimental import pallas as pl
from jax.experimental.pallas import tpu as pltpu
from jax.experimental.pallas import tpu_sc as plsc

NMPAD = 128


NBUF = 3


def _gather_kernel(n_per_w, n_chunk, num_cores, table_hbm, idx_hbm, out_hbm,
                   idx_v, *bufs):
    wid = lax.axis_index("s") * num_cores + lax.axis_index("c")
    base = wid * n_per_w
    nc = n_per_w // n_chunk
    rows = bufs[:NBUF]
    gsem = bufs[NBUF:2 * NBUF]
    ssem = bufs[2 * NBUF:]
    pltpu.sync_copy(idx_hbm.at[pl.ds(base, n_per_w)], idx_v)

    def issue_gather(c):
        buf = c % NBUF
        return pltpu.async_copy(
            table_hbm.at[idx_v.at[pl.ds(c * n_chunk, n_chunk)]],
            rows[buf], gsem[buf])

    gat = [None] * NBUF
    scat = [None] * NBUF
    for c in range(min(NBUF, nc)):
        gat[c % NBUF] = issue_gather(c)
    for c in range(nc):
        buf = c % NBUF
        gat[buf].wait()
        scat[buf] = pltpu.async_copy(
            rows[buf], out_hbm.at[pl.ds(base + c * n_chunk, n_chunk)],
            ssem[buf])
        nxt = c + NBUF
        if nxt < nc:
            scat[buf].wait()
            scat[buf] = None
            gat[buf] = issue_gather(nxt)
    for sc in scat:
        if sc is not None:
            sc.wait()


def _sc_gather(table, idx):
    n = idx.shape[0]
    d = table.shape[1]
    info = plsc.get_sparse_core_info()
    nw = info.num_cores * info.num_subcores
    n_per_w = n // nw
    n_chunk = min(32, n_per_w)
    mesh = plsc.VectorSubcoreMesh(core_axis_name="c", subcore_axis_name="s")
    kern = pl.kernel(
        functools.partial(_gather_kernel, n_per_w, n_chunk, info.num_cores),
        mesh=mesh,
        out_type=jax.ShapeDtypeStruct((n, d), jnp.float32),
        scratch_types=(
            [pltpu.VMEM((n_per_w,), jnp.int32)]
            + [pltpu.VMEM((n_chunk, d), jnp.float32) for _ in range(NBUF)]
            + [pltpu.SemaphoreType.DMA for _ in range(2 * NBUF)]
        ),
    )
    return kern(table, idx)


def _top2_routing(logits, nm):
    blk = logits.shape[0]
    col = lax.broadcasted_iota(jnp.int32, (blk, NMPAD), 1)
    neg = jnp.float32(-1e30)
    ml = jnp.where(col < nm, logits, neg)
    m1 = jnp.max(ml, axis=1, keepdims=True)
    i1 = jnp.min(jnp.where(ml >= m1, col, NMPAD), axis=1, keepdims=True)
    oh1 = col == i1
    ml2 = jnp.where(oh1, neg, ml)
    m2 = jnp.max(ml2, axis=1, keepdims=True)
    i2 = jnp.min(jnp.where(ml2 >= m2, col, NMPAD), axis=1, keepdims=True)
    oh2 = col == i2
    t = jnp.exp(m2 - m1)
    g1 = 1.0 / (1.0 + t)
    g2 = 1.0 - g1
    zero = jnp.float32(0.0)
    gate = jnp.where(oh1, g1, zero) + jnp.where(oh2, g2, zero)
    wm = jnp.where(oh1 | oh2, jnp.float32(1.0), zero)
    return gate, wm


def _mega_kernel(bq, nq, nm, h, xe_ref, wqkv_ref, wg_ref, wo_ref,
                 bo_ref, o_ref, qkvs_ref, gs_ref, wms_ref, acc_ref):
    t = pl.program_id(1)
    cdims = (((1,), (1,)), ((), ()))

    @pl.when(t < nq)
    def _proj():
        xe = xe_ref[0]
        xb = xe.astype(jnp.bfloat16)
        sl = pl.ds(t * bq, bq)
        qkvs_ref[sl, :] = jnp.dot(
            xb, wqkv_ref[...],
            preferred_element_type=jnp.float32).astype(jnp.bfloat16)
        logits = jnp.dot(xe, wg_ref[...], preferred_element_type=jnp.float32)
        gate, wm = _top2_routing(logits, nm)
        gs_ref[sl, :] = gate.astype(jnp.bfloat16)
        wms_ref[sl, :] = wm.astype(jnp.bfloat16)

    @pl.when(t >= nq)
    def _flash():
        u = t - nq
        i = u * 0
        for r in range(1, nq):
            i = i + jnp.where(u >= (r * (r + 1)) // 2, 1, 0)
        j = u - (i * (i + 1)) // 2
        q = qkvs_ref[pl.ds(i * bq, bq), :h]
        gate = gs_ref[pl.ds(i * bq, bq), :]
        ks = qkvs_ref[pl.ds(j * bq, bq), h:2 * h]
        vs = qkvs_ref[pl.ds(j * bq, bq), 2 * h:]
        wms = wms_ref[pl.ds(j * bq, bq), :]

        @pl.when(j < i)
        def _():
            s = lax.dot_general(q, ks, cdims,
                                preferred_element_type=jnp.float32)
            r = lax.dot_general(gate, wms, cdims,
                                preferred_element_type=jnp.float32)
            a = s * r
            pa = jnp.dot(a.astype(jnp.bfloat16), vs,
                         preferred_element_type=jnp.float32)
            acc_ref[...] = jnp.where(j == 0, pa, acc_ref[...] + pa)

        @pl.when(j == i)
        def _():
            s = lax.dot_general(q, ks, cdims,
                                preferred_element_type=jnp.float32)
            r = lax.dot_general(gate, wms, cdims,
                                preferred_element_type=jnp.float32)
            rows = lax.broadcasted_iota(jnp.int32, (bq, bq), 0)
            cols = lax.broadcasted_iota(jnp.int32, (bq, bq), 1)
            a = jnp.where(rows >= cols, s * r, jnp.float32(0.0))
            pa = jnp.dot(a.astype(jnp.bfloat16), vs,
                         preferred_element_type=jnp.float32)
            acc = jnp.where(i == 0, pa, acc_ref[...] + pa)
            o_ref[0] = (jnp.dot(acc.astype(jnp.bfloat16), wo_ref[...],
                                preferred_element_type=jnp.float32)
                        + bo_ref[...])


def kernel(x, emb_table, Wq, Wk, Wv, Wg, Wo, bo):
    b, s = x.shape
    e = emb_table.shape[1]
    h = Wq.shape[1]
    nm = Wg.shape[1]
    o = Wo.shape[1]
    bq = 512
    nq = s // bq
    wgp = jnp.pad(Wg, ((0, 0), (0, NMPAD - nm)))
    ntri = (nq * (nq + 1)) // 2

    def _row(t):
        u = t - nq
        i = u * 0
        for r in range(1, nq):
            i = i + jnp.where(u >= (r * (r + 1)) // 2, 1, 0)
        return i

    kern = pl.pallas_call(
        functools.partial(_mega_kernel, bq, nq, nm, h),
        grid=(b, nq + ntri),
        in_specs=[
            pl.BlockSpec((1, bq, e),
                         lambda b_, t: (b_, jnp.minimum(t, nq - 1), 0)),
            pl.BlockSpec((e, 3 * h), lambda b_, t: (0, 0)),
            pl.BlockSpec((e, NMPAD), lambda b_, t: (0, 0)),
            pl.BlockSpec((h, o), lambda b_, t: (0, 0)),
            pl.BlockSpec((1, o), lambda b_, t: (0, 0)),
        ],
        out_specs=pl.BlockSpec(
            (1, bq, o),
            lambda b_, t: (b_, jnp.where(t < nq, 0, _row(t)), 0)),
        out_shape=jax.ShapeDtypeStruct((b, s, o), jnp.float32),
        scratch_shapes=[
            pltpu.VMEM((s, 3 * h), jnp.bfloat16),
            pltpu.VMEM((s, NMPAD), jnp.bfloat16),
            pltpu.VMEM((s, NMPAD), jnp.bfloat16),
            pltpu.VMEM((bq, h), jnp.float32),
        ],
    )
    idx = x.reshape(-1).astype(jnp.int32)
    xe = _sc_gather(emb_table, idx)
    wqkv = jnp.concatenate([Wq, Wk, Wv], axis=1).astype(jnp.bfloat16)
    out = kern(xe.reshape(b, s, e), wqkv, wgp,
               Wo.astype(jnp.bfloat16), bo.reshape(1, o))
    return out


# speedup vs baseline: 1.2672x; 1.0816x over previous
import functools

import jax
import jax.numpy as jnp
from jax import lax
from jax.experimental import pallas as pl
from jax.experimental.pallas import tpu as pltpu
from jax.experimental.pallas import tpu_sc as plsc

NMPAD = 128


NBUF = 3


def _gather_kernel(n_per_w, n_chunk, num_cores, table_hbm, idx_hbm, out_hbm,
                   idx_v, *bufs):
    wid = lax.axis_index("s") * num_cores + lax.axis_index("c")
    base = wid * n_per_w
    nc = n_per_w // n_chunk
    rows = bufs[:NBUF]
    gsem = bufs[NBUF:2 * NBUF]
    ssem = bufs[2 * NBUF:]
    pltpu.sync_copy(idx_hbm.at[pl.ds(base, n_per_w)], idx_v)

    def issue_gather(c):
        buf = c % NBUF
        return pltpu.async_copy(
            table_hbm.at[idx_v.at[pl.ds(c * n_chunk, n_chunk)]],
            rows[buf], gsem[buf])

    gat = [None] * NBUF
    scat = [None] * NBUF
    for c in range(min(NBUF, nc)):
        gat[c % NBUF] = issue_gather(c)
    for c in range(nc):
        buf = c % NBUF
        gat[buf].wait()
        scat[buf] = pltpu.async_copy(
            rows[buf], out_hbm.at[pl.ds(base + c * n_chunk, n_chunk)],
            ssem[buf])
        nxt = c + NBUF
        if nxt < nc:
            scat[buf].wait()
            scat[buf] = None
            gat[buf] = issue_gather(nxt)
    for sc in scat:
        if sc is not None:
            sc.wait()


def _sc_gather(table, idx):
    n = idx.shape[0]
    d = table.shape[1]
    info = plsc.get_sparse_core_info()
    nw = info.num_cores * info.num_subcores
    n_per_w = n // nw
    n_chunk = min(32, n_per_w)
    mesh = plsc.VectorSubcoreMesh(core_axis_name="c", subcore_axis_name="s")
    kern = pl.kernel(
        functools.partial(_gather_kernel, n_per_w, n_chunk, info.num_cores),
        mesh=mesh,
        out_type=jax.ShapeDtypeStruct((n, d), jnp.float32),
        scratch_types=(
            [pltpu.VMEM((n_per_w,), jnp.int32)]
            + [pltpu.VMEM((n_chunk, d), jnp.float32) for _ in range(NBUF)]
            + [pltpu.SemaphoreType.DMA for _ in range(2 * NBUF)]
        ),
    )
    return kern(table, idx)


def _top2_routing(logits, nm):
    blk = logits.shape[0]
    col = lax.broadcasted_iota(jnp.int32, (blk, NMPAD), 1)
    neg = jnp.float32(-1e30)
    ml = jnp.where(col < nm, logits, neg)
    m1 = jnp.max(ml, axis=1, keepdims=True)
    i1 = jnp.min(jnp.where(ml >= m1, col, NMPAD), axis=1, keepdims=True)
    oh1 = col == i1
    ml2 = jnp.where(oh1, neg, ml)
    m2 = jnp.max(ml2, axis=1, keepdims=True)
    i2 = jnp.min(jnp.where(ml2 >= m2, col, NMPAD), axis=1, keepdims=True)
    oh2 = col == i2
    t = jnp.exp(m2 - m1)
    g1 = 1.0 / (1.0 + t)
    g2 = 1.0 - g1
    zero = jnp.float32(0.0)
    gate = jnp.where(oh1, g1, zero) + jnp.where(oh2, g2, zero)
    wm = jnp.where(oh1 | oh2, jnp.float32(1.0), zero)
    return gate, wm


def _mega_kernel(bq, nq, nm, h, xe_ref, wqkv_ref, wg_ref, wo_ref,
                 bo_ref, o_ref, qkvs_ref, gs_ref, wms_ref):
    t = pl.program_id(1)
    cdims = (((1,), (1,)), ((), ()))

    @pl.when(t < nq)
    def _proj():
        xe = xe_ref[0]
        xb = xe.astype(jnp.bfloat16)
        sl = pl.ds(t * bq, bq)
        qkvs_ref[sl, :] = jnp.dot(
            xb, wqkv_ref[...],
            preferred_element_type=jnp.float32).astype(jnp.bfloat16)
        logits = jnp.dot(xe, wg_ref[...], preferred_element_type=jnp.float32)
        gate, wm = _top2_routing(logits, nm)
        gs_ref[sl, :] = gate.astype(jnp.bfloat16)
        wms_ref[sl, :] = wm.astype(jnp.bfloat16)

    for ic in range(nq):
        @pl.when(t == nq + ic)
        def _row_step(ic=ic):
            w = (ic + 1) * bq
            q = qkvs_ref[pl.ds(ic * bq, bq), :h]
            gate = gs_ref[pl.ds(ic * bq, bq), :]
            ks = qkvs_ref[pl.ds(0, w), h:2 * h]
            vs = qkvs_ref[pl.ds(0, w), 2 * h:]
            wms = wms_ref[pl.ds(0, w), :]
            s = lax.dot_general(q, ks, cdims,
                                preferred_element_type=jnp.float32)
            r = lax.dot_general(gate, wms, cdims,
                                preferred_element_type=jnp.float32)
            rows = lax.broadcasted_iota(jnp.int32, (bq, w), 0)
            cols = lax.broadcasted_iota(jnp.int32, (bq, w), 1)
            a = jnp.where(ic * bq + rows >= cols, s * r, jnp.float32(0.0))
            pa = jnp.dot(a.astype(jnp.bfloat16), vs,
                         preferred_element_type=jnp.float32)
            o_ref[0] = (jnp.dot(pa.astype(jnp.bfloat16), wo_ref[...],
                                preferred_element_type=jnp.float32)
                        + bo_ref[...])


def kernel(x, emb_table, Wq, Wk, Wv, Wg, Wo, bo):
    b, s = x.shape
    e = emb_table.shape[1]
    h = Wq.shape[1]
    nm = Wg.shape[1]
    o = Wo.shape[1]
    bq = 512
    nq = s // bq
    wgp = jnp.pad(Wg, ((0, 0), (0, NMPAD - nm)))
    kern = pl.pallas_call(
        functools.partial(_mega_kernel, bq, nq, nm, h),
        grid=(b, 2 * nq),
        in_specs=[
            pl.BlockSpec((1, bq, e),
                         lambda b_, t: (b_, jnp.minimum(t, nq - 1), 0)),
            pl.BlockSpec((e, 3 * h), lambda b_, t: (0, 0)),
            pl.BlockSpec((e, NMPAD), lambda b_, t: (0, 0)),
            pl.BlockSpec((h, o), lambda b_, t: (0, 0)),
            pl.BlockSpec((1, o), lambda b_, t: (0, 0)),
        ],
        out_specs=pl.BlockSpec(
            (1, bq, o),
            lambda b_, t: (b_, jnp.where(t < nq, 0, t - nq), 0)),
        out_shape=jax.ShapeDtypeStruct((b, s, o), jnp.float32),
        scratch_shapes=[
            pltpu.VMEM((s, 3 * h), jnp.bfloat16),
            pltpu.VMEM((s, NMPAD), jnp.bfloat16),
            pltpu.VMEM((s, NMPAD), jnp.bfloat16),
        ],
    )
    idx = x.reshape(-1).astype(jnp.int32)
    xe = _sc_gather(emb_table, idx)
    wqkv = jnp.concatenate([Wq, Wk, Wv], axis=1).astype(jnp.bfloat16)
    out = kern(xe.reshape(b, s, e), wqkv, wgp,
               Wo.astype(jnp.bfloat16), bo.reshape(1, o))
    return out


# speedup vs baseline: 1.2978x; 1.0241x over previous
import functools

import jax
import jax.numpy as jnp
from jax import lax
from jax.experimental import pallas as pl
from jax.experimental.pallas import tpu as pltpu
from jax.experimental.pallas import tpu_sc as plsc

NMPAD = 128


NBUF = 3


def _gather_kernel(n_per_w, n_chunk, num_cores, table_hbm, idx_hbm, out_hbm,
                   idx_v, *bufs):
    wid = lax.axis_index("s") * num_cores + lax.axis_index("c")
    base = wid * n_per_w
    nc = n_per_w // n_chunk
    rows = bufs[:NBUF]
    gsem = bufs[NBUF:2 * NBUF]
    ssem = bufs[2 * NBUF:]
    pltpu.sync_copy(idx_hbm.at[pl.ds(base, n_per_w)], idx_v)

    def issue_gather(c):
        buf = c % NBUF
        return pltpu.async_copy(
            table_hbm.at[idx_v.at[pl.ds(c * n_chunk, n_chunk)]],
            rows[buf], gsem[buf])

    gat = [None] * NBUF
    scat = [None] * NBUF
    for c in range(min(NBUF, nc)):
        gat[c % NBUF] = issue_gather(c)
    for c in range(nc):
        buf = c % NBUF
        gat[buf].wait()
        scat[buf] = pltpu.async_copy(
            rows[buf], out_hbm.at[pl.ds(base + c * n_chunk, n_chunk)],
            ssem[buf])
        nxt = c + NBUF
        if nxt < nc:
            scat[buf].wait()
            scat[buf] = None
            gat[buf] = issue_gather(nxt)
    for sc in scat:
        if sc is not None:
            sc.wait()


def _sc_gather(table, idx):
    n = idx.shape[0]
    d = table.shape[1]
    info = plsc.get_sparse_core_info()
    nw = info.num_cores * info.num_subcores
    n_per_w = n // nw
    n_chunk = min(32, n_per_w)
    mesh = plsc.VectorSubcoreMesh(core_axis_name="c", subcore_axis_name="s")
    kern = pl.kernel(
        functools.partial(_gather_kernel, n_per_w, n_chunk, info.num_cores),
        mesh=mesh,
        out_type=jax.ShapeDtypeStruct((n, d), jnp.float32),
        scratch_types=(
            [pltpu.VMEM((n_per_w,), jnp.int32)]
            + [pltpu.VMEM((n_chunk, d), jnp.float32) for _ in range(NBUF)]
            + [pltpu.SemaphoreType.DMA for _ in range(2 * NBUF)]
        ),
    )
    return kern(table, idx)


def _top2_routing(logits, nm):
    blk = logits.shape[0]
    col = lax.broadcasted_iota(jnp.int32, (blk, NMPAD), 1)
    neg = jnp.float32(-1e30)
    ml = jnp.where(col < nm, logits, neg)
    m1 = jnp.max(ml, axis=1, keepdims=True)
    i1 = jnp.min(jnp.where(ml >= m1, col, NMPAD), axis=1, keepdims=True)
    oh1 = col == i1
    ml2 = jnp.where(oh1, neg, ml)
    m2 = jnp.max(ml2, axis=1, keepdims=True)
    i2 = jnp.min(jnp.where(ml2 >= m2, col, NMPAD), axis=1, keepdims=True)
    oh2 = col == i2
    t = jnp.exp(m2 - m1)
    g1 = 1.0 / (1.0 + t)
    g2 = 1.0 - g1
    zero = jnp.float32(0.0)
    gate = jnp.where(oh1, g1, zero) + jnp.where(oh2, g2, zero)
    wm = jnp.where(oh1 | oh2, jnp.float32(1.0), zero)
    return gate, wm


def _mega_kernel(bq, nq, nm, h, xe_ref, wqkv_ref, wg_ref, wo_ref,
                 bo_ref, o_ref, qkvs_ref, gs_ref, wms_ref):
    t = pl.program_id(1)
    cdims = (((1,), (1,)), ((), ()))

    @pl.when(t == 0)
    def _proj():
        xe = xe_ref[0]
        xb = xe.astype(jnp.bfloat16)
        qkvs_ref[...] = jnp.dot(
            xb, wqkv_ref[...],
            preferred_element_type=jnp.float32).astype(jnp.bfloat16)
        logits = jnp.dot(xe, wg_ref[...], preferred_element_type=jnp.float32)
        gate, wm = _top2_routing(logits, nm)
        gs_ref[...] = gate.astype(jnp.bfloat16)
        wms_ref[...] = wm.astype(jnp.bfloat16)

    for ic in range(nq):
        @pl.when(t == 1 + ic)
        def _row_step(ic=ic):
            w = (ic + 1) * bq
            q = qkvs_ref[pl.ds(ic * bq, bq), :h]
            gate = gs_ref[pl.ds(ic * bq, bq), :]
            ks = qkvs_ref[pl.ds(0, w), h:2 * h]
            vs = qkvs_ref[pl.ds(0, w), 2 * h:]
            wms = wms_ref[pl.ds(0, w), :]
            s = lax.dot_general(q, ks, cdims,
                                preferred_element_type=jnp.float32)
            r = lax.dot_general(gate, wms, cdims,
                                preferred_element_type=jnp.float32)
            rows = lax.broadcasted_iota(jnp.int32, (bq, w), 0)
            cols = lax.broadcasted_iota(jnp.int32, (bq, w), 1)
            a = jnp.where(ic * bq + rows >= cols, s * r, jnp.float32(0.0))
            pa = jnp.dot(a.astype(jnp.bfloat16), vs,
                         preferred_element_type=jnp.float32)
            o_ref[0] = (jnp.dot(pa.astype(jnp.bfloat16), wo_ref[...],
                                preferred_element_type=jnp.float32)
                        + bo_ref[...])


def kernel(x, emb_table, Wq, Wk, Wv, Wg, Wo, bo):
    b, s = x.shape
    e = emb_table.shape[1]
    h = Wq.shape[1]
    nm = Wg.shape[1]
    o = Wo.shape[1]
    bq = 512
    nq = s // bq
    wgp = jnp.pad(Wg, ((0, 0), (0, NMPAD - nm)))
    kern = pl.pallas_call(
        functools.partial(_mega_kernel, bq, nq, nm, h),
        grid=(b, 1 + nq),
        in_specs=[
            pl.BlockSpec((1, s, e), lambda b_, t: (b_, 0, 0)),
            pl.BlockSpec((e, 3 * h), lambda b_, t: (0, 0)),
            pl.BlockSpec((e, NMPAD), lambda b_, t: (0, 0)),
            pl.BlockSpec((h, o), lambda b_, t: (0, 0)),
            pl.BlockSpec((1, o), lambda b_, t: (0, 0)),
        ],
        out_specs=pl.BlockSpec(
            (1, bq, o),
            lambda b_, t: (b_, jnp.where(t == 0, 0, t - 1), 0)),
        out_shape=jax.ShapeDtypeStruct((b, s, o), jnp.float32),
        scratch_shapes=[
            pltpu.VMEM((s, 3 * h), jnp.bfloat16),
            pltpu.VMEM((s, NMPAD), jnp.bfloat16),
            pltpu.VMEM((s, NMPAD), jnp.bfloat16),
        ],
    )
    idx = x.reshape(-1).astype(jnp.int32)
    xe = _sc_gather(emb_table, idx)
    wqkv = jnp.concatenate([Wq, Wk, Wv], axis=1).astype(jnp.bfloat16)
    out = kern(xe.reshape(b, s, e), wqkv, wgp,
               Wo.astype(jnp.bfloat16), bo.reshape(1, o))
    return out
